# Initial kernel scaffold; baseline (speedup 1.0000x reference)
import jax
import jax.numpy as jnp
from jax.experimental import pallas as pl


def kernel(x, edge_index, edge_attr, W1, b1, g1, be1, W2, b2, g2, be2, Wc1, bc1, Wc2, bc2, Wk1, bk1, Wk2, bk2):
    raise NotImplementedError("write your pallas kernel here")


# speedup vs baseline: 12.1328x; 12.1328x over previous
import functools

import jax
import jax.numpy as jnp
from jax import lax
from jax.experimental import pallas as pl
from jax.experimental.pallas import tpu as pltpu
from jax.experimental.pallas import tpu_sc as plsc

N = 10000
E = 320000
D = 128
H = 128
OUT = 64
EPS = 1e-5

NC = 2
NS = 16
NW = NC * NS
CHUNK = 128
EPW = ((E + NW - 1) // NW + CHUNK - 1) // CHUNK * CHUNK
EPAD = EPW * NW
NCHUNK = EPW // CHUNK
NP = 10240
ROWS_PER_TILE = NP // NS
ROW_SEGS = [128] * 5

_mesh = plsc.VectorSubcoreMesh(core_axis_name="c", subcore_axis_name="s")


@functools.partial(
    pl.kernel,
    mesh=_mesh,
    out_type=jax.ShapeDtypeStruct((NC, NP), jnp.float32),
    scratch_types=[
        pltpu.VMEM((CHUNK,), jnp.int32),
        pltpu.VMEM((CHUNK,), jnp.float32),
        pltpu.VMEM((NP // NS,), jnp.float32),
        pltpu.VMEM_SHARED((NP,), jnp.float32),
    ],
)
def _sc_degree(dst_hbm, w_hbm, degp_hbm, dst_v, w_v, zero_v, deg_sh):
    cid = lax.axis_index("c")
    sid = lax.axis_index("s")
    wid = cid * NS + sid
    seg = NP // NS
    for i in range(seg // 16):
        zero_v[pl.ds(i * 16, 16)] = jnp.zeros((16,), jnp.float32)
    pltpu.sync_copy(zero_v, deg_sh.at[pl.ds(sid * seg, seg)])
    plsc.subcore_barrier()

    def body(j, carry):
        base = wid * EPW + j * CHUNK
        pltpu.sync_copy(dst_hbm.at[pl.ds(base, CHUNK)], dst_v)
        pltpu.sync_copy(w_hbm.at[pl.ds(base, CHUNK)], w_v)
        pltpu.sync_copy(w_v, deg_sh.at[dst_v], add=True)
        return carry

    lax.fori_loop(0, NCHUNK, body, 0)
    plsc.subcore_barrier()
    pltpu.sync_copy(deg_sh.at[pl.ds(sid * seg, seg)],
                    degp_hbm.at[cid, pl.ds(sid * seg, seg)])


@functools.partial(
    pl.kernel,
    mesh=_mesh,
    out_type=jax.ShapeDtypeStruct((NC, NP, H), jnp.float32),
    scratch_types=[
        pltpu.VMEM((CHUNK,), jnp.int32),
        pltpu.VMEM((CHUNK,), jnp.int32),
        pltpu.VMEM((CHUNK,), jnp.float32),
        pltpu.VMEM((CHUNK, H), jnp.float32),
        pltpu.VMEM_SHARED((NP, H), jnp.float32),
        pltpu.SemaphoreType.DMA,
    ],
)
def _sc_aggregate(hhat_hbm, src_hbm, dst_hbm, w_hbm, accp_hbm,
                  src_v, dst_v, w_v, rows_v, acc_sh, sem):
    cid = lax.axis_index("c")
    sid = lax.axis_index("s")
    wid = cid * NS + sid

    def zrow(r, carry):
        for f in range(H // 16):
            rows_v[r, pl.ds(f * 16, 16)] = jnp.zeros((16,), jnp.float32)
        return carry

    lax.fori_loop(0, CHUNK, zrow, 0)
    off = 0
    for segn in ROW_SEGS:
        pltpu.sync_copy(rows_v.at[pl.ds(0, segn)],
                        acc_sh.at[pl.ds(sid * ROWS_PER_TILE + off, segn)])
        off += segn
    plsc.subcore_barrier()

    def body(j, carry):
        base = wid * EPW + j * CHUNK
        pltpu.sync_copy(src_hbm.at[pl.ds(base, CHUNK)], src_v)
        pltpu.sync_copy(dst_hbm.at[pl.ds(base, CHUNK)], dst_v)
        pltpu.sync_copy(w_hbm.at[pl.ds(base, CHUNK)], w_v)
        pltpu.async_copy(hhat_hbm.at[src_v], rows_v, sem).wait()

        def scale(g, c2):
            wv16 = w_v[pl.ds(g * 16, 16)]
            for e in range(16):
                wb = lax.gather(
                    wv16, jnp.full((16, 1), e, jnp.int32),
                    lax.GatherDimensionNumbers(
                        offset_dims=(), collapsed_slice_dims=(0,),
                        start_index_map=(0,)),
                    (1,), mode=lax.GatherScatterMode.PROMISE_IN_BOUNDS)
                r = g * 16 + e
                for f in range(H // 16):
                    sl = pl.ds(f * 16, 16)
                    rows_v[r, sl] = rows_v[r, sl] * wb
            return c2

        lax.fori_loop(0, CHUNK // 16, scale, 0)
        pltpu.sync_copy(rows_v, acc_sh.at[dst_v], add=True)
        return carry

    lax.fori_loop(0, NCHUNK, body, 0)
    plsc.subcore_barrier()
    off = 0
    for segn in ROW_SEGS:
        r0 = sid * ROWS_PER_TILE + off
        pltpu.sync_copy(acc_sh.at[pl.ds(r0, segn)],
                        accp_hbm.at[cid, pl.ds(r0, segn)])
        off += segn


def _tc_prolog_body(x_ref, w1_ref, degp_ref, hhat_ref, dinv_ref):
    deg = degp_ref[0, :N, :] + degp_ref[1, :N, :] + 1.0
    dinv = jnp.where(deg > 0, lax.rsqrt(deg), 0.0)
    h = lax.dot_general(x_ref[...], w1_ref[...],
                        (((1,), (0,)), ((), ())),
                        preferred_element_type=jnp.float32)
    hhat_ref[...] = h * dinv
    dinv_ref[...] = dinv


def _tc_mid_body(accp_ref, hhat1_ref, dinv_ref, w2_ref, b1_ref, g1_ref, be1_ref,
                 hhat2_ref):
    dinv = dinv_ref[...]
    acc = accp_ref[0, :N, :] + accp_ref[1, :N, :]
    pre = dinv * (acc + hhat1_ref[...]) + b1_ref[...]
    s1 = g1_ref[...] * (1.0 / jnp.sqrt(1.0 + EPS))
    z = jnp.maximum(pre * s1 + be1_ref[...], 0.0)
    h2 = lax.dot_general(z, w2_ref[...], (((1,), (0,)), ((), ())),
                         preferred_element_type=jnp.float32)
    hhat2_ref[...] = h2 * dinv


def _tc_epilog_body(accp_ref, hhat2_ref, dinv_ref, b2_ref, g2_ref, be2_ref,
                    wc1_ref, bc1_ref, wc2_ref, bc2_ref,
                    wk1_ref, bk1_ref, wk2_ref, bk2_ref,
                    cls_ref, key_ref, h_ref):
    dinv = dinv_ref[...]
    acc = accp_ref[0, :N, :] + accp_ref[1, :N, :]
    pre = dinv * (acc + hhat2_ref[...]) + b2_ref[...]
    s2 = g2_ref[...] * (1.0 / jnp.sqrt(1.0 + EPS))
    h = jnp.maximum(pre * s2 + be2_ref[...], 0.0)
    h_ref[...] = h

    def mm(a, b):
        return lax.dot_general(a, b, (((1,), (0,)), ((), ())),
                               preferred_element_type=jnp.float32)

    c = jnp.maximum(mm(h, wc1_ref[...]) + bc1_ref[...], 0.0)
    cls_ref[...] = mm(c, wc2_ref[...]) + bc2_ref[...]
    k = jnp.maximum(mm(h, wk1_ref[...]) + bk1_ref[...], 0.0)
    key_ref[...] = mm(k, wk2_ref[...]) + bk2_ref[...]


def _tc_call(body, out_shapes, *args):
    return pl.pallas_call(body, out_shape=out_shapes)(*args)


def kernel(x, edge_index, edge_attr, W1, b1, g1, be1, W2, b2, g2, be2,
           Wc1, bc1, Wc2, bc2, Wk1, bk1, Wk2, bk2):
    src = edge_index[0]
    dst = edge_index[1]
    pad = EPAD - E
    pad_idx = (jnp.arange(pad, dtype=jnp.int32) * 37) % N
    src_p = jnp.concatenate([src, pad_idx])
    dst_p = jnp.concatenate([dst, pad_idx])
    w_p = jnp.concatenate([edge_attr, jnp.zeros((pad,), jnp.float32)])

    degp = _sc_degree(dst_p, w_p)
    degp3 = degp.reshape(NC, NP, 1)

    hhat1, dinv = _tc_call(
        _tc_prolog_body,
        [jax.ShapeDtypeStruct((N, H), jnp.float32),
         jax.ShapeDtypeStruct((N, 1), jnp.float32)],
        x, W1, degp3)

    accp1 = _sc_aggregate(hhat1, src_p, dst_p, w_p)

    hhat2 = _tc_call(
        _tc_mid_body,
        jax.ShapeDtypeStruct((N, H), jnp.float32),
        accp1, hhat1, dinv, W2,
        b1.reshape(1, H), g1.reshape(1, H), be1.reshape(1, H))

    accp2 = _sc_aggregate(hhat2, src_p, dst_p, w_p)

    cls, key, h = _tc_call(
        _tc_epilog_body,
        [jax.ShapeDtypeStruct((N, OUT), jnp.float32),
         jax.ShapeDtypeStruct((N, 1), jnp.float32),
         jax.ShapeDtypeStruct((N, H), jnp.float32)],
        accp2, hhat2, dinv,
        b2.reshape(1, H), g2.reshape(1, H), be2.reshape(1, H),
        Wc1, bc1.reshape(1, H // 2), Wc2, bc2.reshape(1, OUT),
        Wk1, bk1.reshape(1, H // 2), Wk2, bk2.reshape(1, 1))
    return (cls, key, h)
